# baseline (speedup 1.0000x reference)
import functools

import jax
import jax.numpy as jnp
from jax import lax
from jax.experimental import pallas as pl
from jax.experimental.pallas import tpu as pltpu
from jax.experimental.pallas import tpu_sc as plsc

_B = 16384
_V = 53
_D = 128
_NC = 2
_NS = 16
_NW = _NC * _NS
_BPW = _B // _NW
_CHR = 256
_NCHK = _BPW // _CHR


def _table_body(rank_ref, suit_ref, dist_ref, pw_ref, pb_ref,
                rw_ref, rb_ref, sw_ref, sb_ref, dw_ref, db_ref, out_ref):
    cat = jnp.concatenate([rank_ref[...], suit_ref[...], dist_ref[...]],
                          axis=1)
    card = jnp.dot(cat, pw_ref[...],
                   preferred_element_type=jnp.float32) + pb_ref[...]
    out_ref[...] = jnp.zeros((_V, _D), jnp.float32)
    out_ref[:, 0:13] = jnp.dot(card, rw_ref[...],
                               preferred_element_type=jnp.float32) + rb_ref[...]
    out_ref[:, 13:17] = jnp.dot(card, sw_ref[...],
                                preferred_element_type=jnp.float32) + sb_ref[...]
    out_ref[:, 17:69] = jnp.dot(card, dw_ref[...],
                                preferred_element_type=jnp.float32) + db_ref[...]


_table_call = pl.pallas_call(
    _table_body,
    out_shape=jax.ShapeDtypeStruct((_V, _D), jnp.float32),
)


@functools.cache
def _make_gather():
    @functools.partial(
        pl.kernel,
        mesh=plsc.VectorSubcoreMesh(core_axis_name="c", subcore_axis_name="s"),
        out_type=(
            jax.ShapeDtypeStruct((_B, 13), jnp.float32),
            jax.ShapeDtypeStruct((_B, 4), jnp.float32),
            jax.ShapeDtypeStruct((_B, 52), jnp.float32),
        ),
        scratch_types=[
            pltpu.VMEM((_BPW,), jnp.int32),
            pltpu.VMEM((_V, _D), jnp.float32),
            pltpu.VMEM((_CHR, 13), jnp.float32),
            pltpu.VMEM((_CHR, 4), jnp.float32),
            pltpu.VMEM((_CHR, 52), jnp.float32),
            pltpu.SemaphoreType.DMA,
        ],
        compiler_params=pltpu.CompilerParams(needs_layout_passes=False),
    )
    def _gather(table_hbm, idx_hbm, rank_hbm, suit_hbm, dist_hbm,
                idx_v, tab_v, rank_v, suit_v, dist_v, sem):
        wid = lax.axis_index("s") * _NC + lax.axis_index("c")
        base = wid * _BPW
        cp_idx = pltpu.async_copy(idx_hbm.at[pl.ds(base, _BPW)], idx_v, sem)
        pltpu.sync_copy(table_hbm, tab_v)
        cp_idx.wait()
        lane = lax.iota(jnp.int32, 16)

        for chunk in range(_NCHK):
            def body(g, carry):
                rows16 = idx_v[pl.ds(chunk * _CHR + g * 16, 16)]
                orow = g * 16 + lane
                for c in range(13):
                    col = jnp.full((16,), c, jnp.int32)
                    vals = plsc.load_gather(tab_v, [rows16, col])
                    plsc.store_scatter(rank_v, [orow, col], vals)
                for c in range(4):
                    tcol = jnp.full((16,), 13 + c, jnp.int32)
                    col = jnp.full((16,), c, jnp.int32)
                    vals = plsc.load_gather(tab_v, [rows16, tcol])
                    plsc.store_scatter(suit_v, [orow, col], vals)
                for c in range(52):
                    tcol = jnp.full((16,), 17 + c, jnp.int32)
                    col = jnp.full((16,), c, jnp.int32)
                    vals = plsc.load_gather(tab_v, [rows16, tcol])
                    plsc.store_scatter(dist_v, [orow, col], vals)
                return carry

            lax.fori_loop(0, _CHR // 16, body, jnp.int32(0))
            cbase = base + chunk * _CHR
            pltpu.sync_copy(rank_v, rank_hbm.at[pl.ds(cbase, _CHR)])
            pltpu.sync_copy(suit_v, suit_hbm.at[pl.ds(cbase, _CHR)])
            pltpu.sync_copy(dist_v, dist_hbm.at[pl.ds(cbase, _CHR)])

    return _gather


def kernel(card_id, rank_tab, suit_tab, dist_tab, proj_W, proj_b,
           rank_W, rank_b, suit_W, suit_b, dist_W, dist_b):
    table = _table_call(rank_tab, suit_tab, dist_tab,
                        proj_W, proj_b.reshape(1, 16),
                        rank_W, rank_b.reshape(1, 13),
                        suit_W, suit_b.reshape(1, 4),
                        dist_W, dist_b.reshape(1, 52))
    idx = card_id.astype(jnp.int32)
    rank_pred, suit_pred, dist_pred = _make_gather()(table, idx)
    return rank_pred, suit_pred, dist_pred


# speedup vs baseline: 1.1940x; 1.1940x over previous
import functools

import jax
import jax.numpy as jnp
from jax import lax
from jax.experimental import pallas as pl
from jax.experimental.pallas import tpu as pltpu
from jax.experimental.pallas import tpu_sc as plsc

_B = 16384
_V = 53
_D = 128
_NC = 2
_NS = 16
_NW = _NC * _NS
_BPW = _B // _NW
_CHR = 256
_NCHK = _BPW // _CHR


def _table_body(rank_ref, suit_ref, dist_ref, pw_ref, pb_ref,
                rw_ref, rb_ref, sw_ref, sb_ref, dw_ref, db_ref, out_ref):
    cat = jnp.concatenate([rank_ref[...], suit_ref[...], dist_ref[...]],
                          axis=1)
    card = jnp.dot(cat, pw_ref[...],
                   preferred_element_type=jnp.float32) + pb_ref[...]
    cardT = card.T
    out_ref[...] = jnp.zeros((_D, _V), jnp.float32)
    out_ref[0:13, :] = (jnp.dot(rw_ref[...].T, cardT,
                                preferred_element_type=jnp.float32)
                        + rb_ref[...].T)
    out_ref[13:17, :] = (jnp.dot(sw_ref[...].T, cardT,
                                 preferred_element_type=jnp.float32)
                         + sb_ref[...].T)
    out_ref[17:69, :] = (jnp.dot(dw_ref[...].T, cardT,
                                 preferred_element_type=jnp.float32)
                         + db_ref[...].T)


_table_call = pl.pallas_call(
    _table_body,
    out_shape=jax.ShapeDtypeStruct((_D, _V), jnp.float32),
)


@functools.cache
def _make_gather():
    @functools.partial(
        pl.kernel,
        mesh=plsc.VectorSubcoreMesh(core_axis_name="c", subcore_axis_name="s"),
        out_type=(
            jax.ShapeDtypeStruct((_B, 13), jnp.float32),
            jax.ShapeDtypeStruct((_B, 4), jnp.float32),
            jax.ShapeDtypeStruct((_B, 52), jnp.float32),
        ),
        scratch_types=[
            pltpu.VMEM((_BPW,), jnp.int32),
            pltpu.VMEM((_D, _V), jnp.float32),
            pltpu.VMEM((_CHR, 13), jnp.float32),
            pltpu.VMEM((_CHR, 4), jnp.float32),
            pltpu.VMEM((_CHR, 52), jnp.float32),
            pltpu.SemaphoreType.DMA,
        ],
        compiler_params=pltpu.CompilerParams(needs_layout_passes=False),
    )
    def _gather(table_hbm, idx_hbm, rank_hbm, suit_hbm, dist_hbm,
                idx_v, tab_v, rank_v, suit_v, dist_v, sem):
        wid = lax.axis_index("s") * _NC + lax.axis_index("c")
        base = wid * _BPW
        cp_idx = pltpu.async_copy(idx_hbm.at[pl.ds(base, _BPW)], idx_v, sem)
        pltpu.sync_copy(table_hbm, tab_v)
        cp_idx.wait()
        lane = lax.iota(jnp.int32, 16)

        for chunk in range(_NCHK):
            def body(g, carry):
                rows16 = idx_v[pl.ds(chunk * _CHR + g * 16, 16)]
                orow = g * 16 + lane
                for c in range(13):
                    col = jnp.full((16,), c, jnp.int32)
                    vals = plsc.load_gather(tab_v, [col, rows16])
                    plsc.store_scatter(rank_v, [orow, col], vals)
                for c in range(4):
                    tcol = jnp.full((16,), 13 + c, jnp.int32)
                    col = jnp.full((16,), c, jnp.int32)
                    vals = plsc.load_gather(tab_v, [tcol, rows16])
                    plsc.store_scatter(suit_v, [orow, col], vals)
                for c in range(52):
                    tcol = jnp.full((16,), 17 + c, jnp.int32)
                    col = jnp.full((16,), c, jnp.int32)
                    vals = plsc.load_gather(tab_v, [tcol, rows16])
                    plsc.store_scatter(dist_v, [orow, col], vals)
                return carry

            lax.fori_loop(0, _CHR // 16, body, jnp.int32(0))
            cbase = base + chunk * _CHR
            pltpu.sync_copy(rank_v, rank_hbm.at[pl.ds(cbase, _CHR)])
            pltpu.sync_copy(suit_v, suit_hbm.at[pl.ds(cbase, _CHR)])
            pltpu.sync_copy(dist_v, dist_hbm.at[pl.ds(cbase, _CHR)])

    return _gather


def kernel(card_id, rank_tab, suit_tab, dist_tab, proj_W, proj_b,
           rank_W, rank_b, suit_W, suit_b, dist_W, dist_b):
    table = _table_call(rank_tab, suit_tab, dist_tab,
                        proj_W, proj_b.reshape(1, 16),
                        rank_W, rank_b.reshape(1, 13),
                        suit_W, suit_b.reshape(1, 4),
                        dist_W, dist_b.reshape(1, 52))
    idx = card_id.astype(jnp.int32)
    rank_pred, suit_pred, dist_pred = _make_gather()(table, idx)
    return rank_pred, suit_pred, dist_pred


# speedup vs baseline: 1.4483x; 1.2130x over previous
import functools

import jax
import jax.numpy as jnp
from jax import lax
from jax.experimental import pallas as pl
from jax.experimental.pallas import tpu as pltpu
from jax.experimental.pallas import tpu_sc as plsc

_B = 16384
_V = 53
_D = 128
_NC = 2
_NS = 16
_NW = _NC * _NS
_BPW = _B // _NW
_CHR = 256
_NCHK = _BPW // _CHR


def _table_body(rank_ref, suit_ref, dist_ref, pw_ref, pb_ref,
                rw_ref, rb_ref, sw_ref, sb_ref, dw_ref, db_ref, out_ref):
    cat = jnp.concatenate([rank_ref[...], suit_ref[...], dist_ref[...]],
                          axis=1)
    card = jnp.dot(cat, pw_ref[...],
                   preferred_element_type=jnp.float32) + pb_ref[...]
    out_ref[...] = jnp.zeros((_V, _D), jnp.float32)
    out_ref[:, 0:13] = jnp.dot(card, rw_ref[...],
                               preferred_element_type=jnp.float32) + rb_ref[...]
    out_ref[:, 16:20] = jnp.dot(card, sw_ref[...],
                                preferred_element_type=jnp.float32) + sb_ref[...]
    out_ref[:, 32:84] = jnp.dot(card, dw_ref[...],
                                preferred_element_type=jnp.float32) + db_ref[...]


_table_call = pl.pallas_call(
    _table_body,
    out_shape=jax.ShapeDtypeStruct((_V, _D), jnp.float32),
)


@functools.cache
def _make_gather():
    @functools.partial(
        pl.kernel,
        mesh=plsc.VectorSubcoreMesh(core_axis_name="c", subcore_axis_name="s"),
        out_type=(
            jax.ShapeDtypeStruct((_B, 13), jnp.float32),
            jax.ShapeDtypeStruct((_B, 4), jnp.float32),
            jax.ShapeDtypeStruct((_B, 52), jnp.float32),
        ),
        scratch_types=[
            pltpu.VMEM((_BPW,), jnp.int32),
            pltpu.VMEM((_V, _D), jnp.float32),
            pltpu.VMEM((_CHR, 13), jnp.float32),
            pltpu.VMEM((_CHR, 4), jnp.float32),
            pltpu.VMEM((_CHR, 52), jnp.float32),
            pltpu.SemaphoreType.DMA,
        ],
        compiler_params=pltpu.CompilerParams(needs_layout_passes=False),
    )
    def _gather(table_hbm, idx_hbm, rank_hbm, suit_hbm, dist_hbm,
                idx_v, tab_v, rank_v, suit_v, dist_v, sem):
        wid = lax.axis_index("s") * _NC + lax.axis_index("c")
        base = wid * _BPW
        pltpu.sync_copy(idx_hbm.at[pl.ds(base, _BPW)], idx_v)
        pltpu.sync_copy(table_hbm, tab_v)
        lane = lax.iota(jnp.int32, 16)
        lane16 = lane + 16
        lane32 = lane + 32
        lane48 = lane + 48
        lane64 = lane + 64
        lane80 = lane + 80
        m13 = lane < 13
        m4 = lane < 4
        jsplats = [jnp.full((16,), j, jnp.int32) for j in range(16)]

        for chunk in range(_NCHK):
            def body(g, carry):
                g16 = g * 16
                rows16 = idx_v[pl.ds(chunk * _CHR + g16, 16)]
                for j in range(16):
                    r = g16 + j
                    isplat = jnp.take(rows16, jsplats[j])
                    rsplat = jnp.full((16,), r, jnp.int32)
                    plsc.store_scatter(
                        rank_v, [rsplat, lane],
                        plsc.load_gather(tab_v, [isplat, lane]), mask=m13)
                    plsc.store_scatter(
                        suit_v, [rsplat, lane],
                        plsc.load_gather(tab_v, [isplat, lane16]), mask=m4)
                    dist_v[r, pl.ds(0, 16)] = plsc.load_gather(
                        tab_v, [isplat, lane32])
                    dist_v[r, pl.ds(16, 16)] = plsc.load_gather(
                        tab_v, [isplat, lane48])
                    dist_v[r, pl.ds(32, 16)] = plsc.load_gather(
                        tab_v, [isplat, lane64])
                    plsc.store_scatter(
                        dist_v, [rsplat, lane48],
                        plsc.load_gather(tab_v, [isplat, lane80]), mask=m4)
                return carry

            lax.fori_loop(0, _CHR // 16, body, jnp.int32(0))
            cbase = base + chunk * _CHR
            pltpu.sync_copy(rank_v, rank_hbm.at[pl.ds(cbase, _CHR)])
            pltpu.sync_copy(suit_v, suit_hbm.at[pl.ds(cbase, _CHR)])
            pltpu.sync_copy(dist_v, dist_hbm.at[pl.ds(cbase, _CHR)])

    return _gather


def kernel(card_id, rank_tab, suit_tab, dist_tab, proj_W, proj_b,
           rank_W, rank_b, suit_W, suit_b, dist_W, dist_b):
    table = _table_call(rank_tab, suit_tab, dist_tab,
                        proj_W, proj_b.reshape(1, 16),
                        rank_W, rank_b.reshape(1, 13),
                        suit_W, suit_b.reshape(1, 4),
                        dist_W, dist_b.reshape(1, 52))
    idx = card_id.astype(jnp.int32)
    rank_pred, suit_pred, dist_pred = _make_gather()(table, idx)
    return rank_pred, suit_pred, dist_pred
